# baseline (speedup 1.0000x reference)
import functools

import jax
import jax.numpy as jnp
from jax import lax
from jax.experimental import pallas as pl
from jax.experimental.pallas import tpu as pltpu
from jax.experimental.pallas import tpu_sc as plsc

D = 64
DP = 128
B = 1024
T = 512
NC = 2
NS = 16
NW = NC * NS
N = B * T
R_PER_W = N // NW
CH = 256
NCHUNK = R_PER_W // CH
NBUF = 4
LOOK = 2
TP = T // CH
LANES = 16


def _emb_kernel(idx_hbm, table_hbm, pos_hbm, out_hbm,
                idx_v, pos_v, rows0, rows1, rows2, rows3, gsem, osem):
    rows = (rows0, rows1, rows2, rows3)
    wid = lax.axis_index("s") * NC + lax.axis_index("c")
    base = wid * R_PER_W
    pltpu.sync_copy(pos_hbm.at[:, pl.ds(0, D)], pos_v)
    pltpu.sync_copy(idx_hbm.at[pl.ds(base, R_PER_W)], idx_v)

    def issue(i, j):
        pltpu.async_copy(
            table_hbm.at[idx_v.at[pl.ds(i * CH, CH)]], rows[j], gsem.at[j]
        )

    def wait_gather(i, j):
        pltpu.make_async_copy(
            table_hbm.at[idx_v.at[pl.ds(i * CH, CH)]], rows[j], gsem.at[j]
        ).wait()

    def start_out(i, j):
        pltpu.async_copy(
            rows[j],
            out_hbm.at[pl.ds(base + i * CH, CH), pl.ds(0, D)],
            osem.at[j],
        )

    def wait_out(i, j):
        pltpu.make_async_copy(
            rows[j],
            out_hbm.at[pl.ds(base + i * CH, CH), pl.ds(0, D)],
            osem.at[j],
        ).wait()

    for i in range(LOOK):
        issue(i, i % NBUF)

    def group(g, carry):
        for j in range(NBUF):
            i = g * NBUF + j
            j2 = (j + LOOK) % NBUF

            @pl.when(i + LOOK < NCHUNK)
            def _issue_ahead():
                @pl.when(i + LOOK >= NBUF)
                def _wait_buf_free():
                    wait_out(i + LOOK - NBUF, j2)

                issue(i + LOOK, j2)

            wait_gather(i, j)
            po = (i % TP) * CH

            def row_body(rr, c2):
                for u in range(2):
                    r = rr * 2 + u
                    for c in range(D // LANES):
                        sl = pl.ds(c * LANES, LANES)
                        rows[j][r, sl] = rows[j][r, sl] + pos_v[po + r, sl]
                return c2

            lax.fori_loop(0, CH // 2, row_body, 0)
            start_out(i, j)
        return carry

    lax.fori_loop(0, NCHUNK // NBUF, group, 0)

    for j in range(NBUF):
        wait_out(NCHUNK - NBUF + j, j)


def _flatten_idx_tc(x):
    blk = 64

    def body(x_ref, o_ref):
        o_ref[...] = x_ref[...].reshape(blk * T)

    return pl.pallas_call(
        body,
        grid=(B // blk,),
        in_specs=[pl.BlockSpec((blk, T), lambda i: (i, 0))],
        out_specs=pl.BlockSpec((blk * T,), lambda i: (i,)),
        out_shape=jax.ShapeDtypeStruct((N,), jnp.int32),
    )(x)


def kernel(inputs, table, pos):
    idx = _flatten_idx_tc(inputs.astype(jnp.int32))
    pos2d = jnp.pad(pos.reshape(T, D).astype(jnp.float32),
                    ((0, 0), (0, DP - D)))

    mesh = plsc.VectorSubcoreMesh(core_axis_name="c", subcore_axis_name="s")
    run = functools.partial(
        pl.kernel,
        mesh=mesh,
        compiler_params=pltpu.CompilerParams(use_tc_tiling_on_sc=False),
        out_type=jax.ShapeDtypeStruct((N, DP), jnp.float32),
        scratch_types=[
            pltpu.VMEM((R_PER_W,), jnp.int32),
            pltpu.VMEM((T, D), jnp.float32),
            pltpu.VMEM((CH, D), jnp.float32),
            pltpu.VMEM((CH, D), jnp.float32),
            pltpu.VMEM((CH, D), jnp.float32),
            pltpu.VMEM((CH, D), jnp.float32),
            pltpu.SemaphoreType.DMA((NBUF,)),
            pltpu.SemaphoreType.DMA((NBUF,)),
        ],
    )(_emb_kernel)
    out128 = run(idx, table, pos2d)
    return out128[:, :D].reshape(B, T, D)
